# speedup vs baseline: 1.0239x; 1.0239x over previous
import functools

import jax
import jax.numpy as jnp
from jax import lax
from jax.experimental import pallas as pl
from jax.experimental.pallas import tpu as pltpu
from jax.experimental.pallas import tpu_sc as plsc

NODE = 512
FEAT = 512
PAIRS = 16
TASKS = 2 * PAIRS
CHUNK = 128
NCHUNK = NODE // CHUNK
SCALE = 1.0 / (512.0 ** 0.5)


@functools.lru_cache(maxsize=None)
def _sc_gather_fn(total):
    info = plsc.get_sparse_core_info()
    nc = info.num_cores

    @functools.partial(
        pl.kernel,
        mesh=plsc.VectorSubcoreMesh(core_axis_name="c", subcore_axis_name="s"),
        out_type=jax.ShapeDtypeStruct((TASKS, NODE, FEAT), jnp.float32),
        scratch_types=[
            pltpu.VMEM((NCHUNK, CHUNK), jnp.int32),
            pltpu.VMEM((CHUNK, FEAT), jnp.float32),
            pltpu.SemaphoreType.DMA,
        ],
    )
    def gather(features_hbm, idx_hbm, out_hbm, idx_v, rows_v, sem):
        wid = lax.axis_index("s") * nc + lax.axis_index("c")
        pltpu.sync_copy(idx_hbm.at[wid], idx_v)
        for j in range(NCHUNK):
            pltpu.async_copy(features_hbm.at[idx_v.at[j]], rows_v, sem).wait()
            pltpu.sync_copy(rows_v, out_hbm.at[wid, pl.ds(j * CHUNK, CHUNK)])

    return gather


def _tc_body(counts_ref, src_ref, ref_ref, out_ref):
    b = pl.program_id(0)
    s = counts_ref[b, 0]
    r = counts_ref[b, 1]
    acc = lax.dot_general(
        src_ref[0], ref_ref[0],
        (((1,), (1,)), ((), ())),
        preferred_element_type=jnp.float32,
    )
    rows = lax.broadcasted_iota(jnp.int32, (NODE, NODE), 0)
    cols = lax.broadcasted_iota(jnp.int32, (NODE, NODE), 1)
    mask = (rows < s) & (cols < r)
    out_ref[0] = jnp.where(mask, acc * SCALE, 0.0)


_tc_scores = pl.pallas_call(
    _tc_body,
    grid=(PAIRS,),
    in_specs=[
        pl.BlockSpec(memory_space=pltpu.SMEM),
        pl.BlockSpec((1, NODE, FEAT), lambda b: (b, 0, 0)),
        pl.BlockSpec((1, NODE, FEAT), lambda b: (b + PAIRS, 0, 0)),
    ],
    out_specs=pl.BlockSpec((1, NODE, NODE), lambda b: (b, 0, 0)),
    out_shape=jax.ShapeDtypeStruct((PAIRS, NODE, NODE), jnp.float32),
)


def kernel(features, src_ref_counts):
    total = features.shape[0]
    counts = jnp.asarray(src_ref_counts).astype(jnp.int32)
    s = counts[:, 0]
    tot = s + counts[:, 1]
    starts = jnp.cumsum(tot) - tot
    offs = jnp.concatenate([starts, starts + s])
    idx = offs[:, None] + jnp.arange(NODE, dtype=jnp.int32)[None, :]
    idx = jnp.minimum(idx, total - 1).reshape(TASKS, NCHUNK, CHUNK)
    gathered = _sc_gather_fn(total)(features, idx)
    return _tc_scores(counts, gathered, gathered)
